# baseline (speedup 1.0000x reference)
import functools

import jax
import jax.numpy as jnp
from jax import lax
from jax.experimental import pallas as pl
from jax.experimental.pallas import tpu as pltpu
from jax.experimental.pallas import tpu_sc as plsc

B = 16384
F_SPARSE = 26
F_DENSE = 13
VOCAB = 1000000
VPAD = 1001472
WROW = VPAD // 2
L = 16
NC = 2
NS = 16
NW = NC * NS
ROWS = B // NW
NIDX = ROWS * F_SPARSE
NCH = ROWS // L


def _sc_body(xs_hbm, xd_hbm, table_hbm, wrep_hbm, out_hbm,
             xs_v, xd_v, wrep_v, idx_v, vals_v, acc_v, sem):
    wid = lax.axis_index("s") * NC + lax.axis_index("c")
    base = wid * ROWS

    pltpu.sync_copy(xs_hbm.at[:, pl.ds(base, ROWS)], xs_v)
    pltpu.sync_copy(xd_hbm.at[:, pl.ds(base, ROWS)], xd_v)
    pltpu.sync_copy(wrep_hbm, wrep_v)

    def build(f_lo, f_hi):
        for f in range(f_lo, f_hi):
            off = f * WROW
            for j in range(NCH):
                sl = pl.ds(j * L, L)
                idx_v[pl.ds(f * ROWS + j * L, L)] = (
                    lax.shift_right_logical(xs_v[f, sl], 1) + off
                )

    FH = F_SPARSE // 2
    build(0, FH)
    g0 = pltpu.async_copy(
        table_hbm.at[idx_v.at[pl.ds(0, FH * ROWS)]],
        vals_v.at[pl.ds(0, FH * ROWS)], sem)
    build(FH, F_SPARSE)
    g1 = pltpu.async_copy(
        table_hbm.at[idx_v.at[pl.ds(FH * ROWS, (F_SPARSE - FH) * ROWS)]],
        vals_v.at[pl.ds(FH * ROWS, (F_SPARSE - FH) * ROWS)], sem)
    g0.wait()
    g1.wait()

    wk = [wrep_v[pl.ds(k * L, L)] for k in range(F_DENSE)]
    himask = jnp.int32(-65536)
    for j in range(NCH):
        sl = pl.ds(j * L, L)
        acc = None
        for f in range(F_SPARSE):
            w = vals_v[pl.ds(f * ROWS + j * L, L)]
            odd = lax.bitwise_and(xs_v[f, sl], 1) == 1
            bits = jnp.where(
                odd,
                lax.bitwise_and(w, himask),
                lax.shift_left(w, 16),
            )
            v = lax.bitcast_convert_type(bits, jnp.float32)
            acc = v if acc is None else acc + v
        for k in range(F_DENSE):
            acc = acc + xd_v[k, sl] * wk[k]
        acc_v[sl] = 1.0 / (1.0 + jnp.exp(-acc))

    pltpu.sync_copy(acc_v, out_hbm.at[pl.ds(base, ROWS)])


@jax.jit
def kernel(X_sparse, X_dense, lin_table, W):
    xs_t = X_sparse.T
    xd_t = X_dense.T
    b = lax.bitcast_convert_type(lin_table, jnp.int32)
    lo = b[:, 0::2, :]
    hi = b[:, 1::2, :]
    rhalf = jnp.int32(0x8000)
    packed = lax.bitwise_or(
        lax.bitwise_and(hi + rhalf, jnp.int32(-65536)),
        lax.bitwise_and(
            jnp.right_shift(lo + rhalf, jnp.int32(16)), jnp.int32(0xFFFF)
        ),
    )
    table = jnp.pad(
        packed, ((0, 0), (0, WROW - VOCAB // 2), (0, 0))
    ).reshape(-1)
    wrep = jnp.repeat(W.reshape(F_DENSE), L)

    mesh = plsc.VectorSubcoreMesh(core_axis_name="c", subcore_axis_name="s")
    run = pl.kernel(
        _sc_body,
        out_type=jax.ShapeDtypeStruct((B,), jnp.float32),
        mesh=mesh,
        scratch_types=[
            pltpu.VMEM((F_SPARSE, ROWS), jnp.int32),
            pltpu.VMEM((F_DENSE, ROWS), jnp.float32),
            pltpu.VMEM((F_DENSE * L,), jnp.float32),
            pltpu.VMEM((NIDX,), jnp.int32),
            pltpu.VMEM((NIDX,), jnp.int32),
            pltpu.VMEM((ROWS,), jnp.float32),
            pltpu.SemaphoreType.DMA,
        ],
    )
    out = run(xs_t, xd_t, table, wrep)
    return out.reshape(B, 1)


# speedup vs baseline: 44.8526x; 44.8526x over previous
import functools

import jax
import jax.numpy as jnp
from jax import lax
from jax.experimental import pallas as pl
from jax.experimental.pallas import tpu as pltpu
from jax.experimental.pallas import tpu_sc as plsc

B = 16384
F_SPARSE = 26
F_DENSE = 13
VOCAB = 1000000
VPAD = 1000448
L = 16
NC = 2
NS = 16
NW = NC * NS
ROWS = B // NW
NIDX = ROWS * F_SPARSE
NCH = ROWS // L


def _sc_body(xs_hbm, xd_hbm, table_hbm, wrep_hbm, out_hbm,
             xs_v, xd_v, wrep_v, idx_v, vals_v, acc_v, sem):
    wid = lax.axis_index("s") * NC + lax.axis_index("c")
    base = wid * ROWS

    pltpu.sync_copy(xs_hbm.at[:, pl.ds(base, ROWS)], xs_v)
    pltpu.sync_copy(xd_hbm.at[:, pl.ds(base, ROWS)], xd_v)
    pltpu.sync_copy(wrep_hbm, wrep_v)

    FH = F_SPARSE // 2
    def build(f_lo, f_hi):
        for f in range(f_lo, f_hi):
            off = f * VPAD
            for j in range(NCH):
                sl = pl.ds(j * L, L)
                idx_v[pl.ds(f * ROWS + j * L, L)] = xs_v[f, sl] + off

    build(0, FH)
    g0 = pltpu.async_copy(
        table_hbm.at[idx_v.at[pl.ds(0, FH * ROWS)]],
        vals_v.at[pl.ds(0, FH * ROWS)], sem)
    build(FH, F_SPARSE)
    g1 = pltpu.async_copy(
        table_hbm.at[idx_v.at[pl.ds(FH * ROWS, (F_SPARSE - FH) * ROWS)]],
        vals_v.at[pl.ds(FH * ROWS, (F_SPARSE - FH) * ROWS)], sem)
    g0.wait()
    g1.wait()

    wk = [wrep_v[pl.ds(k * L, L)] for k in range(F_DENSE)]
    for j in range(NCH):
        sl = pl.ds(j * L, L)
        acc = vals_v[pl.ds(j * L, L)]
        for f in range(1, F_SPARSE):
            acc = acc + vals_v[pl.ds(f * ROWS + j * L, L)]
        for k in range(F_DENSE):
            acc = acc + xd_v[k, sl] * wk[k]
        acc_v[sl] = 1.0 / (1.0 + jnp.exp(-acc))

    pltpu.sync_copy(acc_v, out_hbm.at[pl.ds(base, ROWS)])


@jax.jit
def kernel(X_sparse, X_dense, lin_table, W):
    xs_t = X_sparse.T
    xd_t = X_dense.T
    table = jnp.pad(lin_table, ((0, 0), (0, VPAD - VOCAB), (0, 0))).reshape(-1)
    wrep = jnp.repeat(W.reshape(F_DENSE), L)

    mesh = plsc.VectorSubcoreMesh(core_axis_name="c", subcore_axis_name="s")
    run = pl.kernel(
        _sc_body,
        out_type=jax.ShapeDtypeStruct((B,), jnp.float32),
        mesh=mesh,
        scratch_types=[
            pltpu.VMEM((F_SPARSE, ROWS), jnp.int32),
            pltpu.VMEM((F_DENSE, ROWS), jnp.float32),
            pltpu.VMEM((F_DENSE * L,), jnp.float32),
            pltpu.VMEM((NIDX,), jnp.int32),
            pltpu.VMEM((NIDX,), jnp.float32),
            pltpu.VMEM((ROWS,), jnp.float32),
            pltpu.SemaphoreType.DMA,
        ],
    )
    out = run(xs_t, xd_t, table, wrep)
    return out.reshape(B, 1)
